# Initial kernel scaffold; baseline (speedup 1.0000x reference)
import jax
import jax.numpy as jnp
from jax.experimental import pallas as pl


def kernel(x, h, edge_indices, batch_size, W1, b1, W2, b2, U1, c1, ln_g, ln_b, U2, c2):
    raise NotImplementedError("write your pallas kernel here")


# speedup vs baseline: 6.1337x; 6.1337x over previous
import functools

import jax
import jax.numpy as jnp
from jax import lax
from jax.experimental import pallas as pl
from jax.experimental.pallas import tpu as pltpu
from jax.experimental.pallas import tpu_sc as plsc

NC = 2
NS = 16
NW = NC * NS
CH = 40
CW = 16


def _sc_dist_kernel(NV, E):
    EPW = E // NW
    mesh = plsc.VectorSubcoreMesh(core_axis_name="c", subcore_axis_name="s")

    @functools.partial(
        pl.kernel,
        out_type=jax.ShapeDtypeStruct((E,), jnp.float32),
        mesh=mesh,
        scratch_types=[
            pltpu.VMEM((NV,), jnp.float32),
            pltpu.VMEM((NV,), jnp.float32),
            pltpu.VMEM((NV,), jnp.float32),
            pltpu.VMEM((EPW,), jnp.int32),
            pltpu.VMEM((EPW,), jnp.int32),
            pltpu.VMEM((EPW,), jnp.float32),
        ],
        compiler_params=pltpu.CompilerParams(needs_layout_passes=False),
    )
    def k1(x0_hbm, x1_hbm, x2_hbm, row_hbm, col_hbm, d2_hbm,
           x0, x1, x2, rows_v, cols_v, d2_v):
        c = lax.axis_index("c")
        s = lax.axis_index("s")
        wid = c * NS + s
        base = wid * EPW
        pltpu.sync_copy(x0_hbm, x0)
        pltpu.sync_copy(x1_hbm, x1)
        pltpu.sync_copy(x2_hbm, x2)
        pltpu.sync_copy(row_hbm.at[pl.ds(base, EPW)], rows_v)
        pltpu.sync_copy(col_hbm.at[pl.ds(base, EPW)], cols_v)

        def body(i, carry):
            sl = pl.ds(i * 16, 16)
            r16 = rows_v[sl]
            c16 = cols_v[sl]
            dx = plsc.load_gather(x0, [r16]) - plsc.load_gather(x0, [c16])
            dy = plsc.load_gather(x1, [r16]) - plsc.load_gather(x1, [c16])
            dz = plsc.load_gather(x2, [r16]) - plsc.load_gather(x2, [c16])
            d2_v[sl] = dx * dx + dy * dy + dz * dz
            return carry

        lax.fori_loop(0, EPW // 16, body, 0)
        pltpu.sync_copy(d2_v, d2_hbm.at[pl.ds(base, EPW)])

    return k1


def _sc_message_kernel(NV, D, E):
    EPW = E // NW
    NCH = EPW // CH
    mesh = plsc.VectorSubcoreMesh(core_axis_name="c", subcore_axis_name="s")

    @functools.partial(
        pl.kernel,
        out_type=jax.ShapeDtypeStruct((E, D), jnp.float32),
        mesh=mesh,
        scratch_types=[
            pltpu.VMEM((CH, D), jnp.float32),
            pltpu.VMEM((CH, D), jnp.float32),
            pltpu.VMEM((CH,), jnp.int32),
            pltpu.SemaphoreType.DMA,
        ],
        compiler_params=pltpu.CompilerParams(needs_layout_passes=False),
    )
    def k3(h_hbm, col_hbm, filt_hbm, msg_hbm, hrows_v, filt_v, cidx_v, sem):
        c = lax.axis_index("c")
        s = lax.axis_index("s")
        wid = c * NS + s
        base = wid * EPW

        def chunk(j, carry):
            cb = base + j * CH
            pltpu.sync_copy(col_hbm.at[pl.ds(cb, CH)], cidx_v)
            gather = pltpu.async_copy(h_hbm.at[cidx_v], hrows_v, sem)
            pltpu.sync_copy(filt_hbm.at[pl.ds(cb, CH)], filt_v)
            gather.wait()

            def mrow(r, carry2):
                for k in range(D // 16):
                    sl = pl.ds(k * 16, 16)
                    hrows_v[r, sl] = hrows_v[r, sl] * filt_v[r, sl]
                return carry2

            lax.fori_loop(0, CH, mrow, 0)
            pltpu.sync_copy(hrows_v, msg_hbm.at[pl.ds(cb, CH)])
            return carry

        lax.fori_loop(0, NCH, chunk, 0)

    return k3


def _tc_filter(d2, W1, b1, W2, b2):
    E = d2.shape[0]
    H = W1.shape[1]
    D = W2.shape[1]
    BE = 4000
    assert E % BE == 0

    def body(d2_ref, w1_ref, b1_ref, w2_ref, b2_ref, out_ref):
        d = jnp.sqrt(d2_ref[:, :])
        hid = d * w1_ref[:, :] + b1_ref[:, :]
        hid = hid * jax.nn.sigmoid(hid)
        out_ref[:, :] = (
            jnp.dot(hid, w2_ref[:, :], preferred_element_type=jnp.float32,
                    precision=lax.Precision.HIGHEST)
            + b2_ref[:, :]
        )

    return pl.pallas_call(
        body,
        grid=(E // BE,),
        in_specs=[
            pl.BlockSpec((BE, 1), lambda i: (i, 0)),
            pl.BlockSpec((1, H), lambda i: (0, 0)),
            pl.BlockSpec((1, H), lambda i: (0, 0)),
            pl.BlockSpec((H, D), lambda i: (0, 0)),
            pl.BlockSpec((1, D), lambda i: (0, 0)),
        ],
        out_specs=pl.BlockSpec((BE, D), lambda i: (i, 0)),
        out_shape=jax.ShapeDtypeStruct((E, D), jnp.float32),
    )(d2.reshape(E, 1), W1, b1.reshape(1, H), W2, b2.reshape(1, D))


def _tc_update(h2d, sums_p, cnt_p, U1, c1, ln_g, ln_b, U2, c2):
    NV, D = h2d.shape
    H = U1.shape[1]
    BN = 1000
    assert NV % BN == 0

    def body(h_ref, sp_ref, cp_ref, u1_ref, c1_ref, g_ref, b_ref, u2_ref,
             c2_ref, out_ref):
        sums = sp_ref[:, :]
        cnt = cp_ref[:, :]
        agg = sums / jnp.maximum(cnt, 1.0)
        hb = h_ref[:, :]
        hid = (
            jnp.dot(hb, u1_ref[0:D, :], preferred_element_type=jnp.float32,
                    precision=lax.Precision.HIGHEST)
            + jnp.dot(agg, u1_ref[D:2 * D, :],
                      preferred_element_type=jnp.float32,
                      precision=lax.Precision.HIGHEST)
            + c1_ref[:, :]
        )
        mu = jnp.mean(hid, axis=1, keepdims=True)
        var = jnp.mean((hid - mu) ** 2, axis=1, keepdims=True)
        hid = (hid - mu) / jnp.sqrt(var + 1e-5) * g_ref[:, :] + b_ref[:, :]
        hid = hid * jax.nn.sigmoid(hid)
        out_ref[:, :] = (
            jnp.dot(hid, u2_ref[:, :], preferred_element_type=jnp.float32,
                    precision=lax.Precision.HIGHEST)
            + c2_ref[:, :]
        )

    return pl.pallas_call(
        body,
        grid=(NV // BN,),
        in_specs=[
            pl.BlockSpec((BN, D), lambda i: (i, 0)),
            pl.BlockSpec((BN, D), lambda i: (i, 0)),
            pl.BlockSpec((BN, 1), lambda i: (i, 0)),
            pl.BlockSpec((2 * D, H), lambda i: (0, 0)),
            pl.BlockSpec((1, H), lambda i: (0, 0)),
            pl.BlockSpec((1, H), lambda i: (0, 0)),
            pl.BlockSpec((1, H), lambda i: (0, 0)),
            pl.BlockSpec((H, D), lambda i: (0, 0)),
            pl.BlockSpec((1, D), lambda i: (0, 0)),
        ],
        out_specs=pl.BlockSpec((BN, D), lambda i: (i, 0)),
        out_shape=jax.ShapeDtypeStruct((NV, D), jnp.float32),
    )(h2d, sums_p, cnt_p, U1, c1.reshape(1, H), ln_g.reshape(1, H),
      ln_b.reshape(1, H), U2, c2.reshape(1, D))


def kernel(x, h, edge_indices, batch_size, W1, b1, W2, b2, U1, c1, ln_g,
           ln_b, U2, c2):
    B, N, D = h.shape
    NV = B * N
    E = edge_indices.shape[1]
    assert E % (NW * CH) == 0 and NV % CH == 0 and (E // NW) % 16 == 0

    xf = x.reshape(NV, 3)
    x0, x1, x2 = xf[:, 0].copy(), xf[:, 1].copy(), xf[:, 2].copy()
    row = edge_indices[0]
    col = edge_indices[1]
    h2d = h.reshape(NV, D)

    d2 = _sc_dist_kernel(NV, E)(x0, x1, x2, row, col)
    filt = _tc_filter(d2, W1, b1, W2, b2)
    msg = _sc_message_kernel(NV, D, E)(h2d, col, filt)
    sums = jax.ops.segment_sum(msg, row, num_segments=NV)
    cnt = jax.ops.segment_sum(jnp.ones((E,), jnp.float32), row,
                              num_segments=NV)
    out2d = _tc_update(h2d, sums, cnt.reshape(NV, 1), U1, c1, ln_g, ln_b,
                       U2, c2)
    return out2d.reshape(B, N, D)
